# baseline (speedup 1.0000x reference)
import functools
import jax
import jax.numpy as jnp
from jax import lax
from jax.experimental import pallas as pl
from jax.experimental.pallas import tpu as pltpu, tpu_sc as plsc

B, S, D = 32, 1280, 768
NLOW = 256
P = 16
LPG = NLOW // P
HPG = (S - NLOW) // P
K = 16


def _fold_lanes(a):
    while a.shape[-1] > 1:
        h = a.shape[-1] // 2
        a = a[..., :h] + a[..., h:]
    return a


def _norm_sum(v):
    ll = v * v
    c = [_fold_lanes(ll[:, i * 256:(i + 1) * 256]) for i in range(3)]
    return (c[0] + c[1]) + c[2]


def _tc_body(x_ref, ln_ref, s_ref, n_ref):
    xb = x_ref[0]
    low = xb[0:NLOW]
    high = xb[NLOW:S]

    nh = jnp.sqrt(_norm_sum(high))
    ln = low / jnp.sqrt(_norm_sum(low))
    hn = high / nh

    lnr = ln.reshape(P, LPG, D)
    a = lnr[:, 0:8] + lnr[:, 8:16]
    a = a[:, 0:4] + a[:, 4:8]
    a = a[:, 0:2] + a[:, 2:4]
    q = (a[:, 0:1] + a[:, 1:2]) * jnp.float32(1.0 / 16.0)

    hnr = hn.reshape(P, HPG, D)
    q16 = q.astype(jnp.bfloat16)
    h16 = hnr.astype(jnp.bfloat16)
    srows = []
    for g in range(P):
        srows.append(lax.dot_general(
            q16[g], h16[g], (((1,), (1,)), ((), ())),
            preferred_element_type=jnp.float32))
    ln_ref[0] = ln
    s_ref[0] = jnp.concatenate(srows, axis=0)
    n_ref[0] = nh.reshape(P, HPG)


def _tc_stage(x):
    return pl.pallas_call(
        _tc_body,
        grid=(B,),
        in_specs=[pl.BlockSpec((1, S, D), lambda b: (b, 0, 0))],
        out_specs=[
            pl.BlockSpec((1, NLOW, D), lambda b: (b, 0, 0)),
            pl.BlockSpec((1, P, HPG), lambda b: (b, 0, 0)),
            pl.BlockSpec((1, P, HPG), lambda b: (b, 0, 0)),
        ],
        out_shape=[
            jax.ShapeDtypeStruct((B, NLOW, D), jnp.float32),
            jax.ShapeDtypeStruct((B, P, HPG), jnp.float32),
            jax.ShapeDtypeStruct((B, P, HPG), jnp.float32),
        ],
    )(x)


def _sc_stage(x2d, scores, norms, ln):
    mesh = plsc.VectorSubcoreMesh(core_axis_name="c", subcore_axis_name="s")

    @functools.partial(
        pl.kernel, mesh=mesh,
        out_type=jax.ShapeDtypeStruct((B, 2 * NLOW, D), jnp.float32),
        scratch_types=[
            pltpu.VMEM((P, HPG), jnp.float32),
            pltpu.VMEM((P, HPG), jnp.float32),
            pltpu.VMEM((K,), jnp.int32),
            pltpu.VMEM((K, D), jnp.float32),
            pltpu.SemaphoreType.DMA,
        ],
    )
    def sc_kernel(x_hbm, s_hbm, n_hbm, ln_hbm, out_hbm,
                  scores_v, norms_v, idx_v, rows_v, sem):
        wid = lax.axis_index("s") * 2 + lax.axis_index("c")
        b = wid
        pltpu.sync_copy(s_hbm.at[b], scores_v)
        pltpu.sync_copy(n_hbm.at[b], norms_v)
        pltpu.sync_copy(ln_hbm.at[b], out_hbm.at[b, pl.ds(0, NLOW)])
        lanes = lax.iota(jnp.int32, K)

        def dgather(v, idx):
            return v.at[idx].get(mode="promise_in_bounds")

        def allred(v, op):
            for sh in (8, 4, 2, 1):
                v = op(v, dgather(v, (lanes + sh) & (K - 1)))
            return v

        def g_body(g, carry):
            sv = [scores_v[g, pl.ds(j * K, K)] for j in range(4)]
            nv = [norms_v[g, pl.ds(j * K, K)] for j in range(4)]
            sel = jnp.zeros((K,), jnp.int32)
            for k in range(K):
                m01 = jnp.maximum(sv[0], sv[1])
                m23 = jnp.maximum(sv[2], sv[3])
                mm = allred(jnp.maximum(m01, m23), jnp.maximum)
                cands = [jnp.where(sv[j] == mm, lanes + j * K, HPG)
                         for j in range(4)]
                c01 = jnp.minimum(cands[0], cands[1])
                c23 = jnp.minimum(cands[2], cands[3])
                pick = allred(jnp.minimum(c01, c23), jnp.minimum)
                sel = jnp.where(lanes == k, pick, sel)
                for j in range(4):
                    sv[j] = jnp.where(lanes + j * K == pick,
                                      jnp.float32(-jnp.inf), sv[j])
            rn_sel = dgather(nv[0], sel & (K - 1))
            for j in range(1, 4):
                rn_sel = jnp.where((sel >> 4) == j,
                                   dgather(nv[j], sel & (K - 1)), rn_sel)
            idx_v[...] = b * S + NLOW + g * HPG + sel
            pltpu.async_copy(x_hbm.at[idx_v], rows_v, sem).wait()

            def k_body(k, c2):
                splat = dgather(rn_sel, jnp.full((K,), k, jnp.int32))
                for j in range(D // K):
                    rows_v[k, pl.ds(j * K, K)] = (
                        rows_v[k, pl.ds(j * K, K)] / splat)
                return c2

            lax.fori_loop(0, K, k_body, 0)
            pltpu.sync_copy(rows_v, out_hbm.at[b, pl.ds(NLOW + g * K, K)])
            return carry

        lax.fori_loop(0, P, g_body, 0)

    return sc_kernel(x2d, scores, norms, ln)


def kernel(x):
    ln, scores, norms = _tc_stage(x)
    return _sc_stage(x.reshape(B * S, D), scores, norms, ln)


# speedup vs baseline: 1.0420x; 1.0420x over previous
import functools
import jax
import jax.numpy as jnp
from jax import lax
from jax.experimental import pallas as pl
from jax.experimental.pallas import tpu as pltpu, tpu_sc as plsc

B, S, D = 32, 1280, 768
NLOW = 256
P = 16
LPG = NLOW // P
HPG = (S - NLOW) // P
K = 16


def _fold_lanes(a):
    while a.shape[-1] > 1:
        h = a.shape[-1] // 2
        a = a[..., :h] + a[..., h:]
    return a


def _norm_sum(v):
    ll = v * v
    c = [_fold_lanes(ll[:, i * 256:(i + 1) * 256]) for i in range(3)]
    return (c[0] + c[1]) + c[2]


def _tc_body(x_ref, ln_ref, s_ref, n_ref):
    xb = x_ref[0]
    low = xb[0:NLOW]
    high = xb[NLOW:S]

    nh = jnp.sqrt(_norm_sum(high))
    ln = low / jnp.sqrt(_norm_sum(low))
    hn = high / nh

    lnr = ln.reshape(P, LPG, D)
    a = lnr[:, 0:8] + lnr[:, 8:16]
    a = a[:, 0:4] + a[:, 4:8]
    a = a[:, 0:2] + a[:, 2:4]
    q = (a[:, 0:1] + a[:, 1:2]) * jnp.float32(1.0 / 16.0)

    hnr = hn.reshape(P, HPG, D)
    q16 = q.astype(jnp.bfloat16)
    h16 = hnr.astype(jnp.bfloat16)
    srows = []
    for g in range(P):
        srows.append(lax.dot_general(
            q16[g], h16[g], (((1,), (1,)), ((), ())),
            preferred_element_type=jnp.float32))
    ln_ref[0] = ln
    s_ref[0] = jnp.concatenate(srows, axis=0)
    n_ref[0] = nh.reshape(P, HPG)


def _tc_stage(x):
    return pl.pallas_call(
        _tc_body,
        grid=(B,),
        in_specs=[pl.BlockSpec((1, S, D), lambda b: (b, 0, 0))],
        out_specs=[
            pl.BlockSpec((1, NLOW, D), lambda b: (b, 0, 0)),
            pl.BlockSpec((1, P, HPG), lambda b: (b, 0, 0)),
            pl.BlockSpec((1, P, HPG), lambda b: (b, 0, 0)),
        ],
        out_shape=[
            jax.ShapeDtypeStruct((B, NLOW, D), jnp.float32),
            jax.ShapeDtypeStruct((B, P, HPG), jnp.float32),
            jax.ShapeDtypeStruct((B, P, HPG), jnp.float32),
        ],
    )(x)


def _sc_stage(x2d, scores, norms, ln):
    mesh = plsc.VectorSubcoreMesh(core_axis_name="c", subcore_axis_name="s")

    @functools.partial(
        pl.kernel, mesh=mesh,
        out_type=jax.ShapeDtypeStruct((B, 2 * NLOW, D), jnp.float32),
        scratch_types=[
            pltpu.VMEM((P, HPG), jnp.float32),
            pltpu.VMEM((P, HPG), jnp.float32),
            pltpu.VMEM((K,), jnp.int32),
            pltpu.VMEM((K, D), jnp.float32),
            pltpu.SemaphoreType.DMA,
            pltpu.SemaphoreType.DMA,
        ],
    )
    def sc_kernel(x_hbm, s_hbm, n_hbm, ln_hbm, out_hbm,
                  scores_v, norms_v, idx_v, rows_v, sem, sem_low):
        wid = lax.axis_index("s") * 2 + lax.axis_index("c")
        b = wid
        pltpu.sync_copy(s_hbm.at[b], scores_v)
        pltpu.sync_copy(n_hbm.at[b], norms_v)
        low_cp = pltpu.async_copy(ln_hbm.at[b], out_hbm.at[b, pl.ds(0, NLOW)],
                                  sem_low)
        lanes = lax.iota(jnp.int32, K)

        def dgather(v, idx):
            return v.at[idx].get(mode="promise_in_bounds")

        def allred(v, op):
            for sh in (8, 4, 2, 1):
                v = op(v, dgather(v, (lanes + sh) & (K - 1)))
            return v

        def g_body(g, carry):
            sv = [scores_v[g, pl.ds(j * K, K)] for j in range(4)]
            nv = [norms_v[g, pl.ds(j * K, K)] for j in range(4)]
            sel = jnp.zeros((K,), jnp.int32)
            for k in range(K):
                m01 = jnp.maximum(sv[0], sv[1])
                m23 = jnp.maximum(sv[2], sv[3])
                mm = allred(jnp.maximum(m01, m23), jnp.maximum)
                cands = [jnp.where(sv[j] == mm, lanes + j * K, HPG)
                         for j in range(4)]
                c01 = jnp.minimum(cands[0], cands[1])
                c23 = jnp.minimum(cands[2], cands[3])
                pick = allred(jnp.minimum(c01, c23), jnp.minimum)
                sel = jnp.where(lanes == k, pick, sel)
                for j in range(4):
                    sv[j] = jnp.where(lanes + j * K == pick,
                                      jnp.float32(-jnp.inf), sv[j])
            rn_sel = dgather(nv[0], sel & (K - 1))
            for j in range(1, 4):
                rn_sel = jnp.where((sel >> 4) == j,
                                   dgather(nv[j], sel & (K - 1)), rn_sel)
            idx_v[...] = b * S + NLOW + g * HPG + sel
            pltpu.async_copy(x_hbm.at[idx_v], rows_v, sem).wait()

            def k_body(k, c2):
                splat = dgather(rn_sel, jnp.full((K,), k, jnp.int32))
                for j in range(D // K):
                    rows_v[k, pl.ds(j * K, K)] = (
                        rows_v[k, pl.ds(j * K, K)] / splat)
                return c2

            lax.fori_loop(0, K, k_body, 0)
            pltpu.sync_copy(rows_v, out_hbm.at[b, pl.ds(NLOW + g * K, K)])
            return carry

        lax.fori_loop(0, P, g_body, 0)
        low_cp.wait()

    return sc_kernel(x2d, scores, norms, ln)


def kernel(x):
    ln, scores, norms = _tc_stage(x)
    return _sc_stage(x.reshape(B * S, D), scores, norms, ln)


# speedup vs baseline: 4.2140x; 4.0441x over previous
import functools
import jax
import jax.numpy as jnp
from jax import lax
from jax.experimental import pallas as pl
from jax.experimental.pallas import tpu as pltpu, tpu_sc as plsc

B, S, D = 32, 1280, 768
NLOW = 256
P = 16
LPG = NLOW // P
HPG = (S - NLOW) // P
K = 16


def _fold_lanes(a):
    while a.shape[-1] > 1:
        h = a.shape[-1] // 2
        a = a[..., :h] + a[..., h:]
    return a


def _norm_sum(v):
    ll = v * v
    c = [_fold_lanes(ll[:, i * 256:(i + 1) * 256]) for i in range(3)]
    return (c[0] + c[1]) + c[2]


def _tc_body(x_ref, ln_ref, s_ref, n_ref):
    xb = x_ref[0]
    low = xb[0:NLOW]
    high = xb[NLOW:S]

    nh = jnp.sqrt(_norm_sum(high))
    ln = low / jnp.sqrt(_norm_sum(low))
    hn = high / nh

    lnr = ln.reshape(P, LPG, D)
    a = lnr[:, 0:8] + lnr[:, 8:16]
    a = a[:, 0:4] + a[:, 4:8]
    a = a[:, 0:2] + a[:, 2:4]
    q = (a[:, 0:1] + a[:, 1:2]) * jnp.float32(1.0 / 16.0)

    hnr = hn.reshape(P, HPG, D)
    q16 = q.astype(jnp.bfloat16)
    h16 = hnr.astype(jnp.bfloat16)
    srows = []
    for g in range(P):
        srows.append(lax.dot_general(
            q16[g], h16[g], (((1,), (1,)), ((), ())),
            preferred_element_type=jnp.float32))
    ln_ref[0] = ln
    s_ref[0] = jnp.concatenate(srows, axis=0)
    n_ref[0] = nh.reshape(P, HPG)


def _tc_stage(x):
    return pl.pallas_call(
        _tc_body,
        grid=(B,),
        in_specs=[pl.BlockSpec((1, S, D), lambda b: (b, 0, 0))],
        out_specs=[
            pl.BlockSpec((1, NLOW, D), lambda b: (b, 0, 0)),
            pl.BlockSpec((1, P, HPG), lambda b: (b, 0, 0)),
            pl.BlockSpec((1, P, HPG), lambda b: (b, 0, 0)),
        ],
        out_shape=[
            jax.ShapeDtypeStruct((B, 2 * NLOW, D), jnp.float32),
            jax.ShapeDtypeStruct((B, P, HPG), jnp.float32),
            jax.ShapeDtypeStruct((B, P, HPG), jnp.float32),
        ],
    )(x)


def _sc_stage(x2d, scores, norms):
    mesh = plsc.VectorSubcoreMesh(core_axis_name="c", subcore_axis_name="s")

    @functools.partial(
        pl.kernel, mesh=mesh,
        out_type=jax.ShapeDtypeStruct((B, NLOW, D), jnp.float32),
        scratch_types=[
            pltpu.VMEM((P, HPG), jnp.float32),
            pltpu.VMEM((P, HPG), jnp.float32),
            pltpu.VMEM((K,), jnp.int32),
            pltpu.VMEM((K, D), jnp.float32),
            pltpu.SemaphoreType.DMA,
        ],
    )
    def sc_kernel(x_hbm, s_hbm, n_hbm, out_hbm,
                  scores_v, norms_v, idx_v, rows_v, sem):
        wid = lax.axis_index("s") * 2 + lax.axis_index("c")
        b = wid
        pltpu.sync_copy(s_hbm.at[b], scores_v)
        pltpu.sync_copy(n_hbm.at[b], norms_v)
        lanes = lax.iota(jnp.int32, K)

        def dgather(v, idx):
            return v.at[idx].get(mode="promise_in_bounds")

        def allred(v, op):
            for sh in (8, 4, 2, 1):
                v = op(v, dgather(v, (lanes + sh) & (K - 1)))
            return v

        def g_body(g, carry):
            sv = [scores_v[g, pl.ds(j * K, K)] for j in range(4)]
            nv = [norms_v[g, pl.ds(j * K, K)] for j in range(4)]
            sel = jnp.zeros((K,), jnp.int32)
            for k in range(K):
                m01 = jnp.maximum(sv[0], sv[1])
                m23 = jnp.maximum(sv[2], sv[3])
                mm = allred(jnp.maximum(m01, m23), jnp.maximum)
                cands = [jnp.where(sv[j] == mm, lanes + j * K, HPG)
                         for j in range(4)]
                c01 = jnp.minimum(cands[0], cands[1])
                c23 = jnp.minimum(cands[2], cands[3])
                pick = allred(jnp.minimum(c01, c23), jnp.minimum)
                sel = jnp.where(lanes == k, pick, sel)
                for j in range(4):
                    sv[j] = jnp.where(lanes + j * K == pick,
                                      jnp.float32(-jnp.inf), sv[j])
            rn_sel = dgather(nv[0], sel & (K - 1))
            for j in range(1, 4):
                rn_sel = jnp.where((sel >> 4) == j,
                                   dgather(nv[j], sel & (K - 1)), rn_sel)
            idx_v[...] = b * S + NLOW + g * HPG + sel
            pltpu.async_copy(x_hbm.at[idx_v], rows_v, sem).wait()

            def k_body(k, c2):
                splat = dgather(rn_sel, jnp.full((K,), k, jnp.int32))
                for j in range(D // K):
                    rows_v[k, pl.ds(j * K, K)] = (
                        rows_v[k, pl.ds(j * K, K)] / splat)
                return c2

            lax.fori_loop(0, K, k_body, 0)
            pltpu.sync_copy(rows_v, out_hbm.at[b, pl.ds(g * K, K)])
            return carry

        lax.fori_loop(0, P, g_body, 0)

    return sc_kernel(x2d, scores, norms)


def kernel(x):
    full, scores, norms = _tc_stage(x)
    high = _sc_stage(x.reshape(B * S, D), scores, norms)
    return lax.dynamic_update_slice(full, high, (0, NLOW, 0))


# speedup vs baseline: 4.4036x; 1.0450x over previous
import functools
import jax
import jax.numpy as jnp
from jax import lax
from jax.experimental import pallas as pl
from jax.experimental.pallas import tpu as pltpu, tpu_sc as plsc

B, S, D = 32, 1280, 768
NLOW = 256
P = 16
LPG = NLOW // P
HPG = (S - NLOW) // P
K = 16


def _fold_lanes(a):
    while a.shape[-1] > 1:
        h = a.shape[-1] // 2
        a = a[..., :h] + a[..., h:]
    return a


def _norm_sum(v):
    ll = v * v
    c = [_fold_lanes(ll[:, i * 256:(i + 1) * 256]) for i in range(3)]
    return (c[0] + c[1]) + c[2]


def _tc_body(x_ref, ln_ref, s_ref, n_ref):
    xb = x_ref[0]
    low = xb[0:NLOW]
    high = xb[NLOW:S]

    nh = jnp.sqrt(_norm_sum(high))
    ln = low / jnp.sqrt(_norm_sum(low))
    hn = high / nh

    lnr = ln.reshape(P, LPG, D)
    a = lnr[:, 0:8] + lnr[:, 8:16]
    a = a[:, 0:4] + a[:, 4:8]
    a = a[:, 0:2] + a[:, 2:4]
    q = (a[:, 0:1] + a[:, 1:2]) * jnp.float32(1.0 / 16.0)

    hnr = hn.reshape(P, HPG, D)
    q16 = q.astype(jnp.bfloat16)
    h16 = hnr.astype(jnp.bfloat16)
    srows = []
    for g in range(P):
        srows.append(lax.dot_general(
            q16[g], h16[g], (((1,), (1,)), ((), ())),
            preferred_element_type=jnp.float32))
    ln_ref[0] = ln
    s_ref[0] = jnp.concatenate(srows, axis=0)
    n_ref[0] = nh.reshape(P, HPG)


def _tc_stage(x):
    return pl.pallas_call(
        _tc_body,
        grid=(B,),
        in_specs=[pl.BlockSpec((1, S, D), lambda b: (b, 0, 0))],
        out_specs=[
            pl.BlockSpec((1, NLOW, D), lambda b: (b, 0, 0)),
            pl.BlockSpec((1, P, HPG), lambda b: (b, 0, 0)),
            pl.BlockSpec((1, P, HPG), lambda b: (b, 0, 0)),
        ],
        out_shape=[
            jax.ShapeDtypeStruct((B, 2 * NLOW, D), jnp.float32),
            jax.ShapeDtypeStruct((B, P, HPG), jnp.float32),
            jax.ShapeDtypeStruct((B, P, HPG), jnp.float32),
        ],
    )(x)


def _sc_stage(x2d, scores, norms):
    mesh = plsc.VectorSubcoreMesh(core_axis_name="c", subcore_axis_name="s")

    @functools.partial(
        pl.kernel, mesh=mesh,
        out_type=jax.ShapeDtypeStruct((B, NLOW, D), jnp.float32),
        scratch_types=[
            pltpu.VMEM((P, HPG), jnp.float32),
            pltpu.VMEM((P, HPG), jnp.float32),
            pltpu.VMEM((K,), jnp.int32),
            pltpu.VMEM((K,), jnp.int32),
            pltpu.VMEM((K, D), jnp.float32),
            pltpu.VMEM((K, D), jnp.float32),
            pltpu.SemaphoreType.DMA,
            pltpu.SemaphoreType.DMA,
        ],
    )
    def sc_kernel(x_hbm, s_hbm, n_hbm, out_hbm,
                  scores_v, norms_v, idx_a, idx_b, rows_a, rows_b,
                  sem_a, sem_b):
        wid = lax.axis_index("s") * 2 + lax.axis_index("c")
        b = wid
        pltpu.sync_copy(s_hbm.at[b], scores_v)
        pltpu.sync_copy(n_hbm.at[b], norms_v)
        lanes = lax.iota(jnp.int32, K)

        def dgather(v, idx):
            return v.at[idx].get(mode="promise_in_bounds")

        def allred(v, op):
            for sh in (8, 4, 2, 1):
                v = op(v, dgather(v, (lanes + sh) & (K - 1)))
            return v

        def topk(g):
            sv = [scores_v[g, pl.ds(j * K, K)] for j in range(4)]
            nv = [norms_v[g, pl.ds(j * K, K)] for j in range(4)]
            sel = jnp.zeros((K,), jnp.int32)
            for k in range(K):
                m01 = jnp.maximum(sv[0], sv[1])
                m23 = jnp.maximum(sv[2], sv[3])
                mm = allred(jnp.maximum(m01, m23), jnp.maximum)
                cands = [jnp.where(sv[j] == mm, lanes + j * K, HPG)
                         for j in range(4)]
                c01 = jnp.minimum(cands[0], cands[1])
                c23 = jnp.minimum(cands[2], cands[3])
                pick = allred(jnp.minimum(c01, c23), jnp.minimum)
                sel = jnp.where(lanes == k, pick, sel)
                for j in range(4):
                    sv[j] = jnp.where(lanes + j * K == pick,
                                      jnp.float32(-jnp.inf), sv[j])
            rn_sel = dgather(nv[0], sel & (K - 1))
            for j in range(1, 4):
                rn_sel = jnp.where((sel >> 4) == j,
                                   dgather(nv[j], sel & (K - 1)), rn_sel)
            return sel, rn_sel

        def drain(g, cp, rows_v, rn_sel):
            cp.wait()

            def k_body(k, c2):
                splat = dgather(rn_sel, jnp.full((K,), k, jnp.int32))
                for j in range(D // K):
                    rows_v[k, pl.ds(j * K, K)] = (
                        rows_v[k, pl.ds(j * K, K)] / splat)
                return c2

            lax.fori_loop(0, K, k_body, 0)
            pltpu.sync_copy(rows_v, out_hbm.at[b, pl.ds(g * K, K)])

        def g2_body(t, carry):
            ga = 2 * t
            gb = 2 * t + 1
            sel_a, rn_a = topk(ga)
            idx_a[...] = b * S + NLOW + ga * HPG + sel_a
            cp_a = pltpu.async_copy(x_hbm.at[idx_a], rows_a, sem_a)
            sel_b, rn_b = topk(gb)
            idx_b[...] = b * S + NLOW + gb * HPG + sel_b
            cp_b = pltpu.async_copy(x_hbm.at[idx_b], rows_b, sem_b)
            drain(ga, cp_a, rows_a, rn_a)
            drain(gb, cp_b, rows_b, rn_b)
            return carry

        lax.fori_loop(0, P // 2, g2_body, 0)

    return sc_kernel(x2d, scores, norms)


def kernel(x):
    full, scores, norms = _tc_stage(x)
    high = _sc_stage(x.reshape(B * S, D), scores, norms)
    return lax.dynamic_update_slice(full, high, (0, NLOW, 0))
